# baseline (speedup 1.0000x reference)
import jax
import jax.numpy as jnp
from jax import lax
from jax.experimental import pallas as pl
from jax.experimental.pallas import tpu as pltpu
from jax.experimental.pallas import tpu_sc as plsc

VOCAB = 1000
PADC = 1024
N_TOK = 51200
NC, NS = 2, 16
NW = NC * NS
PER_W = N_TOK // NW
RC = 16
N_RCHUNK = PER_W // RC
IDX_ROWS = 13


def _lse_body(t_ref, o_ref):
    t = t_ref[...]
    m = jnp.max(t, axis=1, keepdims=True)
    s = jnp.sum(jnp.exp(t - m), axis=1, keepdims=True)
    o_ref[...] = m + jnp.log(s)


def _cost_body(p_ref, o_ref):
    o_ref[...] = jnp.reshape(-jnp.sum(p_ref[...]) * (1.0 / N_TOK), (1, 1))


def _sc_body(table_hbm, idx_hbm, tgt_hbm, lse_hbm,
             out_hbm, part_hbm,
             idx_v, tgt_v, lse_v, rows_pad, rows_out, acc_v,
             semg0, semg1, semo0, semo1):
    wid = lax.axis_index("s") * NC
    base = wid * PER_W
    gsems = (semg0, semg1)
    osems = (semo0, semo1)

    pltpu.sync_copy(idx_hbm.at[wid], idx_v)
    pltpu.sync_copy(tgt_hbm.at[wid], tgt_v)
    pltpu.sync_copy(lse_hbm, lse_v)
    acc_v[0, :] = jnp.zeros((16,), jnp.float32)

    iota16 = lax.iota(jnp.int32, 16)
    tail_r = iota16 // 8
    tail_c = (iota16 % 8) + (VOCAB - 8)

    def idxvec(g):
        return idx_v[g // 8, pl.ds((g % 8) * 16, 16)]

    def tgtvec(g):
        return tgt_v[g // 8, pl.ds((g % 8) * 16, 16)]

    def gstart(g, b):
        pltpu.async_copy(table_hbm.at[idxvec(g)], rows_pad.at[b], gsems[b])

    def gwait(g, b):
        pltpu.make_async_copy(table_hbm.at[idxvec(g)], rows_pad.at[b],
                              gsems[b]).wait()

    def ostart(w, b):
        pltpu.async_copy(rows_out.at[b],
                         out_hbm.at[pl.ds(base + w * (2 * RC), 2 * RC)],
                         osems[b])

    def owait(w, b):
        pltpu.make_async_copy(rows_out.at[b],
                              out_hbm.at[pl.ds(base + w * (2 * RC), 2 * RC)],
                              osems[b]).wait()

    gstart(0, 0)
    gstart(1, 1)

    def body(k, carry):
        for half in range(2):
            w = 2 * k + half
            wb = half
            @pl.when(w >= 2)
            def _():
                owait(w - 2, wb)

            for sub in range(2):
                g = 2 * w + sub
                gwait(g, sub)

                def crow(r, c):
                    for j in range(62):
                        rows_out[wb, sub * RC + r, pl.ds(j * 16, 16)] = \
                            rows_pad[sub, r, pl.ds(j * 16, 16)]
                    return c

                lax.fori_loop(0, RC, crow, 0)
                for rr in range(8):
                    tv = plsc.load_gather(rows_pad.at[sub],
                                          [tail_r + 2 * rr, tail_c])
                    plsc.store_scatter(
                        rows_out.at[wb],
                        [tail_r + 2 * rr + sub * RC, tail_c], tv)

                vals = plsc.load_gather(rows_pad.at[sub], [iota16, tgtvec(g)])
                lsegs = plsc.load_gather(lse_v, [idxvec(g)])
                acc_v[0, :] = acc_v[0, :] + (vals - lsegs)

                @pl.when(g + 2 < N_RCHUNK)
                def _():
                    gstart(g + 2, sub)

            ostart(w, wb)
        return carry

    lax.fori_loop(0, N_RCHUNK // 4, body, 0)
    owait(N_RCHUNK // 2 - 2, 0)
    owait(N_RCHUNK // 2 - 1, 1)
    pltpu.sync_copy(acc_v, part_hbm.at[wid])


def _make_sc_gather():
    mesh = plsc.VectorSubcoreMesh(core_axis_name="c", subcore_axis_name="s",
                                  num_cores=NC, num_subcores=NS)
    return pl.kernel(
        _sc_body,
        out_type=[
            jax.ShapeDtypeStruct((N_TOK, VOCAB), jnp.float32),
            jax.ShapeDtypeStruct((NW, 1, 16), jnp.float32),
        ],
        mesh=mesh,
        compiler_params=pltpu.CompilerParams(use_tc_tiling_on_sc=True,
                                             needs_layout_passes=False),
        scratch_types=[
            pltpu.VMEM((IDX_ROWS, 128), jnp.int32),
            pltpu.VMEM((IDX_ROWS, 128), jnp.int32),
            pltpu.VMEM((VOCAB,), jnp.float32),
            pltpu.VMEM((2, RC, PADC), jnp.float32),
            pltpu.VMEM((2, 2 * RC, VOCAB), jnp.float32),
            pltpu.VMEM((1, 16), jnp.float32),
            pltpu.SemaphoreType.DMA,
            pltpu.SemaphoreType.DMA,
            pltpu.SemaphoreType.DMA,
            pltpu.SemaphoreType.DMA,
        ],
    )


_sc_gather = _make_sc_gather()


def kernel(idx, targets, table):
    idx32 = idx.reshape(-1).astype(jnp.int32)
    tgt32 = targets.reshape(-1).astype(jnp.int32)
    table = table.astype(jnp.float32)

    lse = pl.pallas_call(
        _lse_body,
        out_shape=jax.ShapeDtypeStruct((VOCAB, 1), jnp.float32),
    )(table)

    table_pad = jnp.pad(table, ((0, 0), (0, PADC - VOCAB)))
    pad_tok = IDX_ROWS * 128 - PER_W
    idx_pad = jnp.pad(idx32.reshape(NW, PER_W), ((0, 0), (0, pad_tok)))
    tgt_pad = jnp.pad(tgt32.reshape(NW, PER_W), ((0, 0), (0, pad_tok)))

    logits2, partials = _sc_gather(
        table_pad,
        idx_pad.reshape(NW, IDX_ROWS, 128),
        tgt_pad.reshape(NW, IDX_ROWS, 128),
        lse.reshape(-1),
    )

    cost = pl.pallas_call(
        _cost_body,
        out_shape=jax.ShapeDtypeStruct((1, 1), jnp.float32),
    )(partials.reshape(NW, 16))[0, 0]

    return (logits2, cost)


# speedup vs baseline: 1.0045x; 1.0045x over previous
import jax
import jax.numpy as jnp
from jax import lax
from jax.experimental import pallas as pl
from jax.experimental.pallas import tpu as pltpu
from jax.experimental.pallas import tpu_sc as plsc

VOCAB = 1000
PADC = 1024
N_TOK = 51200
NC, NS = 2, 16
NW = NC * NS
PER_W = N_TOK // NW
RC = 16
N_RCHUNK = PER_W // RC
IDX_ROWS = 13


def _lse_body(t_ref, o_ref):
    t = t_ref[...]
    m = jnp.max(t, axis=1, keepdims=True)
    s = jnp.sum(jnp.exp(t - m), axis=1, keepdims=True)
    o_ref[...] = m + jnp.log(s)


def _cost_body(p_ref, o_ref):
    o_ref[...] = jnp.reshape(-jnp.sum(p_ref[...]) * (1.0 / N_TOK), (1, 1))


def _sc_body(table_hbm, idx_hbm, tgt_hbm, lse_hbm,
             out_hbm, part_hbm,
             idx_v, tgt_v, lse_v, rows_pad, rows_out, acc_v,
             semg0, semg1, semo0, semo1):
    wid = lax.axis_index("s") * NC + lax.axis_index("c")
    base = wid * PER_W
    gsems = (semg0, semg1)
    osems = (semo0, semo1)

    pltpu.sync_copy(idx_hbm.at[wid], idx_v)
    pltpu.sync_copy(tgt_hbm.at[wid], tgt_v)
    pltpu.sync_copy(lse_hbm, lse_v)
    acc_v[0, :] = jnp.zeros((16,), jnp.float32)

    iota16 = lax.iota(jnp.int32, 16)
    tail_r = iota16 // 8
    tail_c = (iota16 % 8) + (VOCAB - 8)

    def idxvec(g):
        return idx_v[g // 8, pl.ds((g % 8) * 16, 16)]

    def tgtvec(g):
        return tgt_v[g // 8, pl.ds((g % 8) * 16, 16)]

    def gstart(g, b):
        pltpu.async_copy(table_hbm.at[idxvec(g)], rows_pad.at[b], gsems[b])

    def gwait(g, b):
        pltpu.make_async_copy(table_hbm.at[idxvec(g)], rows_pad.at[b],
                              gsems[b]).wait()

    def ostart(w, b):
        pltpu.async_copy(rows_out.at[b],
                         out_hbm.at[pl.ds(base + w * (2 * RC), 2 * RC)],
                         osems[b])

    def owait(w, b):
        pltpu.make_async_copy(rows_out.at[b],
                              out_hbm.at[pl.ds(base + w * (2 * RC), 2 * RC)],
                              osems[b]).wait()

    gstart(0, 0)
    gstart(1, 1)

    def body(k, carry):
        for half in range(2):
            w = 2 * k + half
            wb = half
            @pl.when(w >= 2)
            def _():
                owait(w - 2, wb)

            for sub in range(2):
                g = 2 * w + sub
                gwait(g, sub)

                def crow(r, c):
                    for j in range(62):
                        rows_out[wb, sub * RC + r, pl.ds(j * 16, 16)] = \
                            rows_pad[sub, r, pl.ds(j * 16, 16)]
                    return c

                lax.fori_loop(0, RC, crow, 0)
                for rr in range(8):
                    tv = plsc.load_gather(rows_pad.at[sub],
                                          [tail_r + 2 * rr, tail_c])
                    plsc.store_scatter(
                        rows_out.at[wb],
                        [tail_r + 2 * rr + sub * RC, tail_c], tv)

                vals = plsc.load_gather(rows_pad.at[sub], [iota16, tgtvec(g)])
                lsegs = plsc.load_gather(lse_v, [idxvec(g)])
                acc_v[0, :] = acc_v[0, :] + (vals - lsegs)

                @pl.when(g + 2 < N_RCHUNK)
                def _():
                    gstart(g + 2, sub)

            ostart(w, wb)
        return carry

    lax.fori_loop(0, N_RCHUNK // 4, body, 0)
    owait(N_RCHUNK // 2 - 2, 0)
    owait(N_RCHUNK // 2 - 1, 1)
    pltpu.sync_copy(acc_v, part_hbm.at[wid])


def _make_sc_gather():
    mesh = plsc.VectorSubcoreMesh(core_axis_name="c", subcore_axis_name="s",
                                  num_cores=NC, num_subcores=NS)
    return pl.kernel(
        _sc_body,
        out_type=[
            jax.ShapeDtypeStruct((N_TOK, VOCAB), jnp.float32),
            jax.ShapeDtypeStruct((NW, 1, 16), jnp.float32),
        ],
        mesh=mesh,
        compiler_params=pltpu.CompilerParams(use_tc_tiling_on_sc=True,
                                             needs_layout_passes=False,
                                             skip_device_barrier=True),
        scratch_types=[
            pltpu.VMEM((IDX_ROWS, 128), jnp.int32),
            pltpu.VMEM((IDX_ROWS, 128), jnp.int32),
            pltpu.VMEM((VOCAB,), jnp.float32),
            pltpu.VMEM((2, RC, PADC), jnp.float32),
            pltpu.VMEM((2, 2 * RC, VOCAB), jnp.float32),
            pltpu.VMEM((1, 16), jnp.float32),
            pltpu.SemaphoreType.DMA,
            pltpu.SemaphoreType.DMA,
            pltpu.SemaphoreType.DMA,
            pltpu.SemaphoreType.DMA,
        ],
    )


_sc_gather = _make_sc_gather()


def kernel(idx, targets, table):
    idx32 = idx.reshape(-1).astype(jnp.int32)
    tgt32 = targets.reshape(-1).astype(jnp.int32)
    table = table.astype(jnp.float32)

    lse = pl.pallas_call(
        _lse_body,
        out_shape=jax.ShapeDtypeStruct((VOCAB, 1), jnp.float32),
    )(table)

    table_pad = jnp.pad(table, ((0, 0), (0, PADC - VOCAB)))
    pad_tok = IDX_ROWS * 128 - PER_W
    idx_pad = jnp.pad(idx32.reshape(NW, PER_W), ((0, 0), (0, pad_tok)))
    tgt_pad = jnp.pad(tgt32.reshape(NW, PER_W), ((0, 0), (0, pad_tok)))

    logits2, partials = _sc_gather(
        table_pad,
        idx_pad.reshape(NW, IDX_ROWS, 128),
        tgt_pad.reshape(NW, IDX_ROWS, 128),
        lse.reshape(-1),
    )

    cost = pl.pallas_call(
        _cost_body,
        out_shape=jax.ShapeDtypeStruct((1, 1), jnp.float32),
    )(partials.reshape(NW, 16))[0, 0]

    return (logits2, cost)


# speedup vs baseline: 1.0054x; 1.0009x over previous
import jax
import jax.numpy as jnp
from jax import lax
from jax.experimental import pallas as pl
from jax.experimental.pallas import tpu as pltpu
from jax.experimental.pallas import tpu_sc as plsc

VOCAB = 1000
PADC = 1024
N_TOK = 51200
NC, NS = 2, 16
NW = NC * NS
PER_W = N_TOK // NW
RC = 16
N_RCHUNK = PER_W // RC
IDX_ROWS = 13


def _lse_body(t_ref, o_ref):
    t = t_ref[...]
    m = jnp.max(t, axis=1, keepdims=True)
    s = jnp.sum(jnp.exp(t - m), axis=1, keepdims=True)
    o_ref[...] = m + jnp.log(s)


def _cost_body(p_ref, o_ref):
    o_ref[...] = jnp.reshape(-jnp.sum(p_ref[...]) * (1.0 / N_TOK), (1, 1))


def _sc_body(table_hbm, idx_hbm, tgt_hbm, lse_hbm,
             out_hbm, part_hbm,
             idx_v, tgt_v, lse_v, rows_pad, rows_out, acc_v,
             semg0, semg1, semo0, semo1):
    wid = lax.axis_index("s") * NC + lax.axis_index("c")
    base = wid * PER_W
    gsems = (semg0, semg1)
    osems = (semo0, semo1)

    pltpu.sync_copy(idx_hbm.at[wid], idx_v)
    pltpu.sync_copy(tgt_hbm.at[wid], tgt_v)
    pltpu.sync_copy(lse_hbm, lse_v)
    acc_v[0, :] = jnp.zeros((16,), jnp.float32)

    iota16 = lax.iota(jnp.int32, 16)
    tail_r = iota16 // 8
    tail_c = (iota16 % 8) + (VOCAB - 8)

    def idxvec(g):
        return idx_v[g // 8, pl.ds((g % 8) * 16, 16)]

    def tgtvec(g):
        return tgt_v[g // 8, pl.ds((g % 8) * 16, 16)]

    def gstart(g, b):
        pltpu.async_copy(table_hbm.at[idxvec(g)], rows_pad.at[b], gsems[b])

    def gwait(g, b):
        pltpu.make_async_copy(table_hbm.at[idxvec(g)], rows_pad.at[b],
                              gsems[b]).wait()

    def ostart(w, b):
        pltpu.async_copy(rows_out.at[b],
                         out_hbm.at[pl.ds(base + w * (2 * RC), 2 * RC)],
                         osems[b])

    def owait(w, b):
        pltpu.make_async_copy(rows_out.at[b],
                              out_hbm.at[pl.ds(base + w * (2 * RC), 2 * RC)],
                              osems[b]).wait()

    gstart(0, 0)
    gstart(1, 1)

    def body(k, carry):
        for half in range(2):
            w = 2 * k + half
            wb = half
            @pl.when(w >= 2)
            def _():
                owait(w - 2, wb)

            for sub in range(2):
                g = 2 * w + sub
                gwait(g, sub)

                def crow(r, c):
                    for j in range(62):
                        rows_out[wb, sub * RC + r, pl.ds(j * 16, 16)] = \
                            rows_pad[sub, r, pl.ds(j * 16, 16)]
                    return c

                lax.fori_loop(0, RC, crow, 0)
                for rr in range(8):
                    tv = plsc.load_gather(rows_pad.at[sub],
                                          [tail_r + 2 * rr, tail_c])
                    plsc.store_scatter(
                        rows_out.at[wb],
                        [tail_r + 2 * rr + sub * RC, tail_c], tv)

                vals = plsc.load_gather(rows_pad.at[sub], [iota16, tgtvec(g)])
                lsegs = plsc.load_gather(lse_v, [idxvec(g)])
                acc_v[0, :] = acc_v[0, :] + (vals - lsegs)

                @pl.when(g + 2 < N_RCHUNK)
                def _():
                    gstart(g + 2, sub)

            ostart(w, wb)
        return carry

    lax.fori_loop(0, N_RCHUNK // 4, body, 0)
    owait(N_RCHUNK // 2 - 2, 0)
    owait(N_RCHUNK // 2 - 1, 1)
    pltpu.sync_copy(acc_v, part_hbm.at[wid])


def _make_sc_gather():
    mesh = plsc.VectorSubcoreMesh(core_axis_name="c", subcore_axis_name="s",
                                  num_cores=NC, num_subcores=NS)
    return pl.kernel(
        _sc_body,
        out_type=[
            jax.ShapeDtypeStruct((N_TOK, VOCAB), jnp.float32),
            jax.ShapeDtypeStruct((NW, 1, 16), jnp.float32),
        ],
        mesh=mesh,
        compiler_params=pltpu.CompilerParams(use_tc_tiling_on_sc=True,
                                             needs_layout_passes=False),
        scratch_types=[
            pltpu.VMEM((IDX_ROWS, 128), jnp.int32),
            pltpu.VMEM((IDX_ROWS, 128), jnp.int32),
            pltpu.VMEM((VOCAB,), jnp.float32),
            pltpu.VMEM((2, RC, PADC), jnp.float32),
            pltpu.VMEM((2, 2 * RC, VOCAB), jnp.float32),
            pltpu.VMEM((1, 16), jnp.float32),
            pltpu.SemaphoreType.DMA,
            pltpu.SemaphoreType.DMA,
            pltpu.SemaphoreType.DMA,
            pltpu.SemaphoreType.DMA,
        ],
    )


_sc_gather = _make_sc_gather()


def kernel(idx, targets, table):
    idx32 = idx.reshape(-1).astype(jnp.int32)
    tgt32 = targets.reshape(-1).astype(jnp.int32)
    table = table.astype(jnp.float32)

    lse = pl.pallas_call(
        _lse_body,
        out_shape=jax.ShapeDtypeStruct((VOCAB, 1), jnp.float32),
    )(table)

    table_pad = jnp.pad(table, ((0, 0), (0, PADC - VOCAB)))
    pad_tok = IDX_ROWS * 128 - PER_W
    idx_pad = jnp.pad(idx32.reshape(NW, PER_W), ((0, 0), (0, pad_tok)))
    tgt_pad = jnp.pad(tgt32.reshape(NW, PER_W), ((0, 0), (0, pad_tok)))

    logits2, partials = _sc_gather(
        table_pad,
        idx_pad.reshape(NW, IDX_ROWS, 128),
        tgt_pad.reshape(NW, IDX_ROWS, 128),
        lse.reshape(-1),
    )

    cost = pl.pallas_call(
        _cost_body,
        out_shape=jax.ShapeDtypeStruct((1, 1), jnp.float32),
    )(partials.reshape(NW, 16))[0, 0]

    return (logits2, cost)
